# baseline (speedup 1.0000x reference)
import functools

import jax
import jax.numpy as jnp
from jax import lax
from jax.experimental import pallas as pl
from jax.experimental.pallas import tpu as pltpu, tpu_sc as plsc

_K = 8
_NC = 2
_NS = 16
_NW = _NC * _NS


def _knn_body(N, K, R, xr_ref, xT_ref, wuv_ref, idx_ref, uv_ref):
    b = pl.program_id(0)
    xr = xr_ref[0]
    xT = xT_ref[0]
    T = jax.lax.dot_general(xr, xT, (((1,), (0,)), ((), ())),
                            preferred_element_type=jnp.float32)
    s = jnp.sum(xT * xT, axis=0, keepdims=True)
    vals = 2.0 * T - s
    mx1 = jnp.max(vals, axis=1, keepdims=True)
    mx2 = jnp.max(jnp.where(vals == mx1, -jnp.inf, vals), axis=1,
                  keepdims=True)
    vals = vals - mx2
    bits = jax.lax.bitcast_convert_type(vals, jnp.int32)
    imin = jnp.int32(-2147483648)
    key = jnp.where(bits < 0, imin - bits, bits)
    col = jax.lax.broadcasted_iota(jnp.int32, (R, N), 1)
    packed = (key & jnp.int32(~127)) | (jnp.int32(127) - (col & jnp.int32(127)))
    kid = jax.lax.broadcasted_iota(jnp.int32, (R, K), 1)
    nt = N // 128
    cands = []
    for t in range(nt):
        p = packed[:, t * 128:(t + 1) * 128]
        ctile = jnp.zeros((R, K), dtype=jnp.int32)
        for j in range(K):
            mx = jnp.max(p, axis=1, keepdims=True)
            ctile = jnp.where(kid == j, mx, ctile)
            p = jnp.where(p == mx, imin, p)
        cands.append(ctile)
    cand = jnp.concatenate(cands, axis=1)
    M = nt * K
    slotc = jax.lax.broadcasted_iota(jnp.int32, (R, M), 1)
    idx = jnp.zeros((R, K), dtype=jnp.int32)
    for j in range(K):
        mx = jnp.max(cand, axis=1, keepdims=True)
        am = jnp.min(jnp.where(cand == mx, slotc, jnp.int32(M)),
                     axis=1, keepdims=True)
        gj = ((am // K) * 128) + jnp.int32(127) - (mx & jnp.int32(127))
        idx = jnp.where(kid == j, gj, idx)
        cand = jnp.where(slotc == am, imin, cand)
    idx_ref[0] = idx + b * N
    uv_ref[0] = jnp.dot(xr, wuv_ref[...], preferred_element_type=jnp.float32)


def _knn_uv(inputs, Wuv, R=256):
    B, N, C = inputs.shape
    xT = jnp.transpose(inputs, (0, 2, 1))
    return pl.pallas_call(
        functools.partial(_knn_body, N, _K, R),
        grid=(B, N // R),
        in_specs=[
            pl.BlockSpec((1, R, C), lambda b, r: (b, r, 0)),
            pl.BlockSpec((1, C, N), lambda b, r: (b, 0, 0)),
            pl.BlockSpec((C, 2 * C), lambda b, r: (0, 0)),
        ],
        out_specs=[
            pl.BlockSpec((1, R, _K), lambda b, r: (b, r, 0)),
            pl.BlockSpec((1, R, 2 * C), lambda b, r: (b, r, 0)),
        ],
        out_shape=[
            jax.ShapeDtypeStruct((B, N, _K), jnp.int32),
            jax.ShapeDtypeStruct((B, N, 2 * C), jnp.float32),
        ],
    )(inputs, xT, Wuv)


def _gather_reduce(uv, gidx):
    BN, C2 = uv.shape
    C = C2 // 2
    pts_per_w = BN // _NW
    P = 16
    n_chunks = pts_per_w // P
    mesh = plsc.VectorSubcoreMesh(core_axis_name="c", subcore_axis_name="s")

    @functools.partial(
        pl.kernel, mesh=mesh,
        out_type=[
            jax.ShapeDtypeStruct((BN, C), jnp.float32),
            jax.ShapeDtypeStruct((_NW, 8, 16), jnp.float32),
        ],
        scratch_types=[
            pltpu.VMEM((P * _K,), jnp.int32),
            pltpu.VMEM((P * _K, C2), jnp.float32),
            pltpu.VMEM((P, C2), jnp.float32),
            pltpu.VMEM((P, C), jnp.float32),
            pltpu.VMEM((8, 16), jnp.float32),
            pltpu.SemaphoreType.DMA,
        ],
    )
    def sc_kernel(uv_hbm, idx_hbm, m_hbm, parts_hbm,
                  idx_v, rows_v, v_v, m_v, acc_v, sem):
        wid = lax.axis_index("s") * _NC + lax.axis_index("c")
        base_pt = wid * pts_per_w
        for i in range(8):
            acc_v[i] = jnp.zeros((16,), jnp.float32)

        def chunk_body(ch, carry):
            pt0 = base_pt + ch * P
            pltpu.sync_copy(idx_hbm.at[pl.ds(pt0 * _K, P * _K)], idx_v)
            pltpu.async_copy(uv_hbm.at[idx_v], rows_v, sem).wait()
            pltpu.sync_copy(uv_hbm.at[pl.ds(pt0, P)], v_v)
            for p in range(P):
                for g in range(4):
                    sl = pl.ds(g * 16, 16)
                    x0 = rows_v[p * _K, sl]
                    mx = x0
                    s1 = x0
                    s2 = x0 * x0
                    for r in range(1, _K):
                        xx = rows_v[p * _K + r, sl]
                        mx = jnp.maximum(mx, xx)
                        s1 = s1 + xx
                        s2 = s2 + xx * xx
                    vv = v_v[p, pl.ds(C + g * 16, 16)]
                    m_v[p, sl] = mx + vv
                    acc_v[g] = acc_v[g] + s1 + 8.0 * vv
                    acc_v[4 + g] = acc_v[4 + g] + s2 + 2.0 * vv * s1 \
                        + 8.0 * (vv * vv)
            pltpu.sync_copy(m_v, m_hbm.at[pl.ds(pt0, P)])
            return carry

        lax.fori_loop(0, n_chunks, chunk_body, 0)
        pltpu.sync_copy(acc_v, parts_hbm.at[wid])

    return sc_kernel(uv, gidx)


def _epilogue_body(cnt, m_ref, parts_ref, g_ref, b_ref, o_ref):
    C = g_ref.shape[1]
    col = jnp.sum(parts_ref[...], axis=0, keepdims=True)
    mean = col[:, :C] * (1.0 / cnt)
    var = col[:, C:] * (1.0 / cnt) - mean * mean
    sc = g_ref[...] * jax.lax.rsqrt(var + 1e-3)
    sh = b_ref[...] - mean * sc
    z = m_ref[...] * sc + sh
    o_ref[...] = jnp.where(z >= 0.0, z, 0.2 * z)


def kernel(inputs, W, gamma, beta):
    B, N, C = inputs.shape
    K = _K
    W1 = W[:, :C]
    W2 = W[:, C:]
    Wuv = jnp.concatenate([W1.T, (W2 - W1).T], axis=1)
    idx, uv = _knn_uv(inputs, Wuv)
    gidx = idx.reshape(-1)
    uv = uv.reshape(B * N, 2 * C)

    m, parts = _gather_reduce(uv, gidx)

    cnt = float(B * N * K)
    R = 2048
    grid = (B * N) // R
    out = pl.pallas_call(
        functools.partial(_epilogue_body, cnt),
        grid=(grid,),
        in_specs=[
            pl.BlockSpec((R, C), lambda i: (i, 0)),
            pl.BlockSpec((_NW, 2 * C), lambda i: (0, 0)),
            pl.BlockSpec((1, C), lambda i: (0, 0)),
            pl.BlockSpec((1, C), lambda i: (0, 0)),
        ],
        out_specs=pl.BlockSpec((R, C), lambda i: (i, 0)),
        out_shape=jax.ShapeDtypeStruct((B * N, C), jnp.float32),
    )(m, parts.reshape(_NW, 2 * C), gamma.reshape(1, C), beta.reshape(1, C))
    return out.reshape(B, N, C)


# speedup vs baseline: 5.9853x; 5.9853x over previous
import functools

import jax
import jax.numpy as jnp
from jax import lax
from jax.experimental import pallas as pl
from jax.experimental.pallas import tpu as pltpu, tpu_sc as plsc

_K = 8
_NC = 2
_NS = 16
_NW = _NC * _NS


def _knn_body(N, K, R, xr_ref, xT_ref, wuv_ref, idx_ref, uv_ref):
    b = pl.program_id(0)
    xr = xr_ref[0]
    xT = xT_ref[0]
    T = jax.lax.dot_general(xr, xT, (((1,), (0,)), ((), ())),
                            preferred_element_type=jnp.float32)
    s = jnp.sum(xT * xT, axis=0, keepdims=True)
    vals = 2.0 * T - s
    colp = (jnp.float32(N - 1) -
            jax.lax.broadcasted_iota(jnp.int32, (R, N), 1).astype(jnp.float32))
    kidf = jax.lax.broadcasted_iota(jnp.int32, (R, K), 1).astype(jnp.float32)
    idx_f = jnp.zeros((R, K), dtype=jnp.float32)
    for j in range(K):
        mx = jnp.max(vals, axis=1, keepdims=True)
        m = vals == mx
        am = jnp.max(jnp.where(m, colp, jnp.float32(-1.0)), axis=1,
                     keepdims=True)
        idx_f = jnp.where(kidf == float(j), jnp.float32(N - 1) - am, idx_f)
        vals = jnp.where(m, -jnp.inf, vals)
    idx_ref[0] = idx_f.astype(jnp.int32) + b * N
    uv_ref[0] = jnp.dot(xr, wuv_ref[...], preferred_element_type=jnp.float32)


def _knn_uv(inputs, Wuv, R=256):
    B, N, C = inputs.shape
    xT = jnp.transpose(inputs, (0, 2, 1))
    return pl.pallas_call(
        functools.partial(_knn_body, N, _K, R),
        grid=(B, N // R),
        in_specs=[
            pl.BlockSpec((1, R, C), lambda b, r: (b, r, 0)),
            pl.BlockSpec((1, C, N), lambda b, r: (b, 0, 0)),
            pl.BlockSpec((C, 2 * C), lambda b, r: (0, 0)),
        ],
        out_specs=[
            pl.BlockSpec((1, R, _K), lambda b, r: (b, r, 0)),
            pl.BlockSpec((1, R, 2 * C), lambda b, r: (b, r, 0)),
        ],
        out_shape=[
            jax.ShapeDtypeStruct((B, N, _K), jnp.int32),
            jax.ShapeDtypeStruct((B, N, 2 * C), jnp.float32),
        ],
    )(inputs, xT, Wuv)


def _gather_reduce(uv, gidx):
    BN, C2 = uv.shape
    C = C2 // 2
    pts_per_w = BN // _NW
    P = 16
    n_chunks = pts_per_w // P
    mesh = plsc.VectorSubcoreMesh(core_axis_name="c", subcore_axis_name="s")

    @functools.partial(
        pl.kernel, mesh=mesh,
        out_type=[
            jax.ShapeDtypeStruct((BN, C), jnp.float32),
            jax.ShapeDtypeStruct((_NW, 8, 16), jnp.float32),
        ],
        scratch_types=[
            pltpu.VMEM((P * _K,), jnp.int32),
            pltpu.VMEM((P * _K, C2), jnp.float32),
            pltpu.VMEM((P, C2), jnp.float32),
            pltpu.VMEM((P, C), jnp.float32),
            pltpu.VMEM((8, 16), jnp.float32),
            pltpu.SemaphoreType.DMA,
        ],
    )
    def sc_kernel(uv_hbm, idx_hbm, m_hbm, parts_hbm,
                  idx_v, rows_v, v_v, m_v, acc_v, sem):
        wid = lax.axis_index("s") * _NC + lax.axis_index("c")
        base_pt = wid * pts_per_w
        for i in range(8):
            acc_v[i] = jnp.zeros((16,), jnp.float32)

        def chunk_body(ch, carry):
            pt0 = base_pt + ch * P
            pltpu.sync_copy(idx_hbm.at[pl.ds(pt0 * _K, P * _K)], idx_v)
            pltpu.async_copy(uv_hbm.at[idx_v], rows_v, sem).wait()
            pltpu.sync_copy(uv_hbm.at[pl.ds(pt0, P)], v_v)
            for p in range(P):
                for g in range(4):
                    sl = pl.ds(g * 16, 16)
                    x0 = rows_v[p * _K, sl]
                    mx = x0
                    s1 = x0
                    s2 = x0 * x0
                    for r in range(1, _K):
                        xx = rows_v[p * _K + r, sl]
                        mx = jnp.maximum(mx, xx)
                        s1 = s1 + xx
                        s2 = s2 + xx * xx
                    vv = v_v[p, pl.ds(C + g * 16, 16)]
                    m_v[p, sl] = mx + vv
                    acc_v[g] = acc_v[g] + s1 + 8.0 * vv
                    acc_v[4 + g] = acc_v[4 + g] + s2 + 2.0 * vv * s1 \
                        + 8.0 * (vv * vv)
            pltpu.sync_copy(m_v, m_hbm.at[pl.ds(pt0, P)])
            return carry

        lax.fori_loop(0, n_chunks, chunk_body, 0)
        pltpu.sync_copy(acc_v, parts_hbm.at[wid])

    return sc_kernel(uv, gidx)


def _epilogue_body(cnt, m_ref, parts_ref, g_ref, b_ref, o_ref):
    C = g_ref.shape[1]
    col = jnp.sum(parts_ref[...], axis=0, keepdims=True)
    mean = col[:, :C] * (1.0 / cnt)
    var = col[:, C:] * (1.0 / cnt) - mean * mean
    sc = g_ref[...] * jax.lax.rsqrt(var + 1e-3)
    sh = b_ref[...] - mean * sc
    z = m_ref[...] * sc + sh
    o_ref[...] = jnp.where(z >= 0.0, z, 0.2 * z)


def kernel(inputs, W, gamma, beta):
    B, N, C = inputs.shape
    K = _K
    W1 = W[:, :C]
    W2 = W[:, C:]
    Wuv = jnp.concatenate([W1.T, (W2 - W1).T], axis=1)
    ms, parts_list = [], []
    for b in range(B):
        idx_b, uv_b = _knn_uv(inputs[b:b + 1], Wuv)
        m_b, parts_b = _gather_reduce(uv_b.reshape(N, 2 * C),
                                      idx_b.reshape(-1))
        ms.append(m_b)
        parts_list.append(parts_b)
    m = jnp.concatenate(ms, axis=0)
    parts = jnp.concatenate(parts_list, axis=0)

    cnt = float(B * N * K)
    R = 2048
    grid = (B * N) // R
    out = pl.pallas_call(
        functools.partial(_epilogue_body, cnt),
        grid=(grid,),
        in_specs=[
            pl.BlockSpec((R, C), lambda i: (i, 0)),
            pl.BlockSpec((B * _NW, 2 * C), lambda i: (0, 0)),
            pl.BlockSpec((1, C), lambda i: (0, 0)),
            pl.BlockSpec((1, C), lambda i: (0, 0)),
        ],
        out_specs=pl.BlockSpec((R, C), lambda i: (i, 0)),
        out_shape=jax.ShapeDtypeStruct((B * N, C), jnp.float32),
    )(m, parts.reshape(B * _NW, 2 * C), gamma.reshape(1, C),
      beta.reshape(1, C))
    return out.reshape(B, N, C)


# speedup vs baseline: 6.4871x; 1.0838x over previous
import functools

import jax
import jax.numpy as jnp
from jax import lax
from jax.experimental import pallas as pl
from jax.experimental.pallas import tpu as pltpu, tpu_sc as plsc

_K = 8
_NC = 2
_NS = 16
_NW = _NC * _NS


def _knn_body(N, K, R, xr_ref, xT_ref, wuv_ref, idx_ref, uv_ref):
    b = pl.program_id(0)
    xr = xr_ref[0]
    xT = xT_ref[0]
    T = jax.lax.dot_general(xr, xT, (((1,), (0,)), ((), ())),
                            preferred_element_type=jnp.float32)
    s = jnp.sum(xT * xT, axis=0, keepdims=True)
    vals = 2.0 * T - s
    colp = (jnp.float32(N - 1) -
            jax.lax.broadcasted_iota(jnp.int32, (R, N), 1).astype(jnp.float32))
    kidf = jax.lax.broadcasted_iota(jnp.int32, (R, K), 1).astype(jnp.float32)
    r = pl.program_id(1)
    selfp = (jnp.float32(N - 1) - jnp.float32(R) * r.astype(jnp.float32) -
             jax.lax.broadcasted_iota(jnp.int32, (R, 1), 0).astype(jnp.float32))
    idx_f = jnp.where(kidf == 0.0,
                      jnp.broadcast_to(jnp.float32(N - 1) - selfp, (R, K)),
                      jnp.zeros((R, K), dtype=jnp.float32))
    vals = jnp.where(colp == selfp, -jnp.inf, vals)
    for j in range(1, K):
        mx = jnp.max(vals, axis=1, keepdims=True)
        m = vals == mx
        am = jnp.max(jnp.where(m, colp, jnp.float32(-1.0)), axis=1,
                     keepdims=True)
        idx_f = jnp.where(kidf == float(j), jnp.float32(N - 1) - am, idx_f)
        vals = jnp.where(m, -jnp.inf, vals)
    idx_ref[0] = idx_f.astype(jnp.int32) + b * N
    uv_ref[0] = jnp.dot(xr, wuv_ref[...], preferred_element_type=jnp.float32)


def _knn_uv(inputs, Wuv, R=256):
    B, N, C = inputs.shape
    xT = jnp.transpose(inputs, (0, 2, 1))
    return pl.pallas_call(
        functools.partial(_knn_body, N, _K, R),
        grid=(B, N // R),
        in_specs=[
            pl.BlockSpec((1, R, C), lambda b, r: (b, r, 0)),
            pl.BlockSpec((1, C, N), lambda b, r: (b, 0, 0)),
            pl.BlockSpec((C, 2 * C), lambda b, r: (0, 0)),
        ],
        out_specs=[
            pl.BlockSpec((1, R, _K), lambda b, r: (b, r, 0)),
            pl.BlockSpec((1, R, 2 * C), lambda b, r: (b, r, 0)),
        ],
        out_shape=[
            jax.ShapeDtypeStruct((B, N, _K), jnp.int32),
            jax.ShapeDtypeStruct((B, N, 2 * C), jnp.float32),
        ],
    )(inputs, xT, Wuv)


def _gather_reduce(uv, gidx):
    BN, C2 = uv.shape
    C = C2 // 2
    pts_per_w = BN // _NW
    P = 16
    n_chunks = pts_per_w // P
    mesh = plsc.VectorSubcoreMesh(core_axis_name="c", subcore_axis_name="s")

    @functools.partial(
        pl.kernel, mesh=mesh,
        out_type=[
            jax.ShapeDtypeStruct((BN, C), jnp.float32),
            jax.ShapeDtypeStruct((_NW, 8, 16), jnp.float32),
        ],
        scratch_types=[
            pltpu.VMEM((P * _K,), jnp.int32),
            pltpu.VMEM((P * _K, C2), jnp.float32),
            pltpu.VMEM((P, C2), jnp.float32),
            pltpu.VMEM((P, C), jnp.float32),
            pltpu.VMEM((8, 16), jnp.float32),
            pltpu.SemaphoreType.DMA,
        ],
    )
    def sc_kernel(uv_hbm, idx_hbm, m_hbm, parts_hbm,
                  idx_v, rows_v, v_v, m_v, acc_v, sem):
        wid = lax.axis_index("s") * _NC + lax.axis_index("c")
        base_pt = wid * pts_per_w
        for i in range(8):
            acc_v[i] = jnp.zeros((16,), jnp.float32)

        def chunk_body(ch, carry):
            pt0 = base_pt + ch * P
            pltpu.sync_copy(idx_hbm.at[pl.ds(pt0 * _K, P * _K)], idx_v)
            pltpu.async_copy(uv_hbm.at[idx_v], rows_v, sem).wait()
            pltpu.sync_copy(uv_hbm.at[pl.ds(pt0, P)], v_v)
            for p in range(P):
                for g in range(4):
                    sl = pl.ds(g * 16, 16)
                    x0 = rows_v[p * _K, sl]
                    mx = x0
                    s1 = x0
                    s2 = x0 * x0
                    for r in range(1, _K):
                        xx = rows_v[p * _K + r, sl]
                        mx = jnp.maximum(mx, xx)
                        s1 = s1 + xx
                        s2 = s2 + xx * xx
                    vv = v_v[p, pl.ds(C + g * 16, 16)]
                    m_v[p, sl] = mx + vv
                    acc_v[g] = acc_v[g] + s1 + 8.0 * vv
                    acc_v[4 + g] = acc_v[4 + g] + s2 + 2.0 * vv * s1 \
                        + 8.0 * (vv * vv)
            pltpu.sync_copy(m_v, m_hbm.at[pl.ds(pt0, P)])
            return carry

        lax.fori_loop(0, n_chunks, chunk_body, 0)
        pltpu.sync_copy(acc_v, parts_hbm.at[wid])

    return sc_kernel(uv, gidx)


def _epilogue_body(cnt, m_ref, parts_ref, g_ref, b_ref, o_ref):
    C = g_ref.shape[1]
    col = jnp.sum(parts_ref[...], axis=0, keepdims=True)
    mean = col[:, :C] * (1.0 / cnt)
    var = col[:, C:] * (1.0 / cnt) - mean * mean
    sc = g_ref[...] * jax.lax.rsqrt(var + 1e-3)
    sh = b_ref[...] - mean * sc
    z = m_ref[...] * sc + sh
    o_ref[...] = jnp.where(z >= 0.0, z, 0.2 * z)


def kernel(inputs, W, gamma, beta):
    B, N, C = inputs.shape
    K = _K
    W1 = W[:, :C]
    W2 = W[:, C:]
    Wuv = jnp.concatenate([W1.T, (W2 - W1).T], axis=1)
    knn = [_knn_uv(inputs[b:b + 1], Wuv) for b in range(B)]
    ms, parts_list = [], []
    for idx_b, uv_b in knn:
        m_b, parts_b = _gather_reduce(uv_b.reshape(N, 2 * C),
                                      idx_b.reshape(-1))
        ms.append(m_b)
        parts_list.append(parts_b)
    m = jnp.concatenate(ms, axis=0)
    parts = jnp.concatenate(parts_list, axis=0)

    cnt = float(B * N * K)
    R = 2048
    grid = (B * N) // R
    out = pl.pallas_call(
        functools.partial(_epilogue_body, cnt),
        grid=(grid,),
        in_specs=[
            pl.BlockSpec((R, C), lambda i: (i, 0)),
            pl.BlockSpec((B * _NW, 2 * C), lambda i: (0, 0)),
            pl.BlockSpec((1, C), lambda i: (0, 0)),
            pl.BlockSpec((1, C), lambda i: (0, 0)),
        ],
        out_specs=pl.BlockSpec((R, C), lambda i: (i, 0)),
        out_shape=jax.ShapeDtypeStruct((B * N, C), jnp.float32),
    )(m, parts.reshape(B * _NW, 2 * C), gamma.reshape(1, C),
      beta.reshape(1, C))
    return out.reshape(B, N, C)
